# Initial kernel scaffold; baseline (speedup 1.0000x reference)
import jax
import jax.numpy as jnp
from jax.experimental import pallas as pl


def kernel(X, edge_index, edge_weight, W_in, b_in, W_mp1, W_mp2, W_out, b_out):
    raise NotImplementedError("write your pallas kernel here")


# speedup vs baseline: 3.8352x; 3.8352x over previous
import functools

import jax
import jax.numpy as jnp
from jax import lax
from jax.experimental import pallas as pl
from jax.experimental.pallas import tpu as pltpu
from jax.experimental.pallas import tpu_sc as plsc

N = 10000
E = 320000
D = 128
HID = 128

NC = 2
NS = 16
NW = NC * NS
EPW = E // NW
CH = 80
NCHUNK = EPW // CH
NROWCH = N // CH


def _mm_in_body(x_ref, w_ref, b_ref, o_ref):
    acc = jnp.dot(x_ref[...], w_ref[...], preferred_element_type=jnp.float32)
    o_ref[...] = jnp.maximum(acc + b_ref[...], 0.0)


def _combine_body(p_ref, o_ref):
    o_ref[...] = p_ref[0] + p_ref[1]


def _mm_out_body(ah_ref, p_ref, w1_ref, w2_ref, wo_ref, bo_ref, o_ref):
    a2h = p_ref[0] + p_ref[1]
    h2 = jnp.maximum(
        jnp.dot(ah_ref[...], w1_ref[...], preferred_element_type=jnp.float32)
        + jnp.dot(a2h, w2_ref[...], preferred_element_type=jnp.float32),
        0.0,
    )
    o_ref[...] = jnp.dot(h2, wo_ref[...], preferred_element_type=jnp.float32) + bo_ref[...]


def _spmm_sc(src, dst, w, table):
    mesh = plsc.VectorSubcoreMesh(core_axis_name="c", subcore_axis_name="s")

    @functools.partial(
        pl.kernel,
        mesh=mesh,
        out_type=jax.ShapeDtypeStruct((NC, N, D), jnp.float32),
        scratch_types=[
            pltpu.VMEM_SHARED((N, D), jnp.float32),
            pltpu.VMEM((CH,), jnp.int32),
            pltpu.VMEM((CH,), jnp.int32),
            pltpu.VMEM((CH,), jnp.float32),
            pltpu.VMEM((CH, D), jnp.float32),
            pltpu.SemaphoreType.DMA,
        ],
    )
    def spmm(src_hbm, dst_hbm, w_hbm, table_hbm, out_hbm, acc_sh, srcv, dstv, wv, rows, sem):
        c = lax.axis_index("c")
        s = lax.axis_index("s")
        wid = c * NS + s

        def zrow(r, _):
            for f in range(D // 16):
                rows[r, pl.ds(f * 16, 16)] = jnp.zeros((16,), jnp.float32)
            return 0
        lax.fori_loop(0, CH, zrow, 0)

        def zcopy(k, _):
            j = s + k * NS
            @pl.when(j < NROWCH)
            def _():
                pltpu.sync_copy(rows, acc_sh.at[pl.ds(j * CH, CH)])
            return 0
        lax.fori_loop(0, (NROWCH + NS - 1) // NS, zcopy, 0)
        plsc.subcore_barrier()

        base = wid * EPW

        def chunk(j, _):
            off = base + j * CH
            pltpu.sync_copy(src_hbm.at[pl.ds(off, CH)], srcv)
            pltpu.sync_copy(dst_hbm.at[pl.ds(off, CH)], dstv)
            pltpu.sync_copy(w_hbm.at[pl.ds(off, CH)], wv)
            pltpu.async_copy(table_hbm.at[srcv], rows, sem).wait()

            def scale(g, _):
                w16 = wv[pl.ds(g * 16, 16)]
                for jj in range(16):
                    we = w16[jj]
                    e = g * 16 + jj
                    for f in range(D // 16):
                        sl = pl.ds(f * 16, 16)
                        rows[e, sl] = rows[e, sl] * we
                return 0
            lax.fori_loop(0, CH // 16, scale, 0)

            pltpu.sync_copy(rows, acc_sh.at[dstv], add=True)
            return 0

        lax.fori_loop(0, NCHUNK, chunk, 0)
        plsc.subcore_barrier()

        def dump(k, _):
            j = s + k * NS
            @pl.when(j < NROWCH)
            def _():
                pltpu.sync_copy(
                    acc_sh.at[pl.ds(j * CH, CH)],
                    out_hbm.at[c, pl.ds(j * CH, CH)],
                )
            return 0
        lax.fori_loop(0, (NROWCH + NS - 1) // NS, dump, 0)

    return spmm(src, dst, w, table)


def kernel(X, edge_index, edge_weight, W_in, b_in, W_mp1, W_mp2, W_out, b_out):
    src = edge_index[0]
    dst = edge_index[1]
    b_in2 = b_in.reshape(1, HID)
    b_out2 = b_out.reshape(1, 1)

    RB = 1000

    H1 = pl.pallas_call(
        _mm_in_body,
        grid=(N // RB,),
        in_specs=[
            pl.BlockSpec((RB, D), lambda i: (i, 0)),
            pl.BlockSpec((D, HID), lambda i: (0, 0)),
            pl.BlockSpec((1, HID), lambda i: (0, 0)),
        ],
        out_specs=pl.BlockSpec((RB, HID), lambda i: (i, 0)),
        out_shape=jax.ShapeDtypeStruct((N, HID), jnp.float32),
    )(X, W_in, b_in2)

    AHp = _spmm_sc(src, dst, edge_weight, H1)

    AH = pl.pallas_call(
        _combine_body,
        grid=(N // RB,),
        in_specs=[pl.BlockSpec((NC, RB, HID), lambda i: (0, i, 0))],
        out_specs=pl.BlockSpec((RB, HID), lambda i: (i, 0)),
        out_shape=jax.ShapeDtypeStruct((N, HID), jnp.float32),
    )(AHp)

    A2Hp = _spmm_sc(src, dst, edge_weight, AH)

    out = pl.pallas_call(
        _mm_out_body,
        grid=(N // RB,),
        in_specs=[
            pl.BlockSpec((RB, HID), lambda i: (i, 0)),
            pl.BlockSpec((NC, RB, HID), lambda i: (0, i, 0)),
            pl.BlockSpec((HID, HID), lambda i: (0, 0)),
            pl.BlockSpec((HID, HID), lambda i: (0, 0)),
            pl.BlockSpec((HID, 1), lambda i: (0, 0)),
            pl.BlockSpec((1, 1), lambda i: (0, 0)),
        ],
        out_specs=pl.BlockSpec((RB, 1), lambda i: (i, 0)),
        out_shape=jax.ShapeDtypeStruct((N, 1), jnp.float32),
    )(AH, A2Hp, W_mp1, W_mp2, W_out, b_out2)

    return out
